# baseline (speedup 1.0000x reference)
import jax
import jax.numpy as jnp
from jax.experimental import pallas as pl


def _masked_matvec_kernel(x_ref, m_ref, w_ref, b_ref, o_ref):
    x = x_ref[...]
    w = w_ref[...]
    y = jax.lax.dot_general(
        w, x, (((1,), (1,)), ((), ())), preferred_element_type=jnp.float32
    )
    y = y[0] + b_ref[0, 0]
    o_ref[0, 0, :] = jnp.where(m_ref[0, 0, :] > 0, y, 0.0)


def kernel(embs, masks, W, b):
    B, N, D = embs.shape
    R = B * N
    ROWS = 32768
    G = R // ROWS

    x = embs.reshape(R, D)
    m = masks.reshape(R).astype(jnp.float32).reshape(G, 1, ROWS)
    b_arr = b.reshape(1, 1).astype(jnp.float32)

    out = pl.pallas_call(
        _masked_matvec_kernel,
        grid=(G,),
        in_specs=[
            pl.BlockSpec((ROWS, D), lambda i: (i, 0)),
            pl.BlockSpec((1, 1, ROWS), lambda i: (i, 0, 0)),
            pl.BlockSpec((1, D), lambda i: (0, 0)),
            pl.BlockSpec((1, 1), lambda i: (0, 0)),
        ],
        out_specs=pl.BlockSpec((1, 1, ROWS), lambda i: (i, 0, 0)),
        out_shape=jax.ShapeDtypeStruct((G, 1, ROWS), jnp.float32),
    )(x, m, W.astype(jnp.float32), b_arr)

    return out.reshape(B, N)


# speedup vs baseline: 1.0374x; 1.0374x over previous
import jax
import jax.numpy as jnp
from jax.experimental import pallas as pl

_K = 4
_ROWS = 4096


def _masked_matvec_kernel(*refs):
    x_refs = refs[:_K]
    m_ref, w_ref, b_ref, o_ref = refs[_K:]
    w = w_ref[...]
    for k in range(_K):
        x = x_refs[k][...]
        y = jax.lax.dot_general(
            w, x, (((1,), (1,)), ((), ())), preferred_element_type=jnp.float32
        )
        y = y[0] + b_ref[0, 0]
        o_ref[k, 0, :] = jnp.where(m_ref[k, 0, :] > 0, y, 0.0)


def kernel(embs, masks, W, b):
    B, N, D = embs.shape
    R = B * N
    NB = R // _ROWS
    G = NB // _K

    x = embs.reshape(R, D)
    m = masks.reshape(R).astype(jnp.float32).reshape(NB, 1, _ROWS)
    b_arr = b.reshape(1, 1).astype(jnp.float32)

    def x_spec(k):
        return pl.BlockSpec((_ROWS, D), lambda i, k=k: (i * _K + k, 0))

    out = pl.pallas_call(
        _masked_matvec_kernel,
        grid=(G,),
        in_specs=[x_spec(k) for k in range(_K)]
        + [
            pl.BlockSpec((_K, 1, _ROWS), lambda i: (i, 0, 0)),
            pl.BlockSpec((1, D), lambda i: (0, 0)),
            pl.BlockSpec((1, 1), lambda i: (0, 0)),
        ],
        out_specs=pl.BlockSpec((_K, 1, _ROWS), lambda i: (i, 0, 0)),
        out_shape=jax.ShapeDtypeStruct((NB, 1, _ROWS), jnp.float32),
    )(*([x] * _K), m, W.astype(jnp.float32), b_arr)

    return out.reshape(B, N)
